# baseline (speedup 1.0000x reference)
import functools

import jax
import jax.numpy as jnp
from jax.experimental import pallas as pl

FEATURE_DIMS = (20000, 5000, 3000)
UNMASKED = (500, 200, 100)
GRID = 20000
MASK = 512
H = 64
BM = 256


def _stage_a_body(x_ref, w_ref, out_ref):
    out_ref[...] = jax.lax.dot_general(
        x_ref[...], w_ref[...], (((1,), (0,)), ((), ())),
        preferred_element_type=jnp.float32)


def _stage_a(x2d, w):
    fd = x2d.shape[1]
    return pl.pallas_call(
        _stage_a_body,
        grid=(MASK // BM,),
        in_specs=[
            pl.BlockSpec((BM, fd), lambda m: (m, 0)),
            pl.BlockSpec((fd, H), lambda m: (0, 0)),
        ],
        out_specs=pl.BlockSpec((BM, H), lambda m: (m, 0)),
        out_shape=jax.ShapeDtypeStruct((MASK, H), jnp.float32),
    )(x2d, w)


def _stage_b_body(pre0_ref, pre1_ref, pre2_ref, bcat_ref, wo0_ref, bo0_ref,
                  wo1_ref, bo1_ref, wo2t_ref, scores_ref, v_ref):
    prec = jnp.concatenate(
        [pre0_ref[...], pre1_ref[...], pre2_ref[...]], axis=1) + bcat_ref[...]
    m = prec > 0.0
    h = jnp.maximum(prec, 0.0)

    a0 = jax.lax.dot_general(h, wo0_ref[...], (((1,), (0,)), ((), ())),
                             preferred_element_type=jnp.float32) + bo0_ref[...]
    m0 = a0 > 0.0
    h0 = jnp.maximum(a0, 0.0)

    a1 = jax.lax.dot_general(h0, wo1_ref[...], (((1,), (0,)), ((), ())),
                             preferred_element_type=jnp.float32) + bo1_ref[...]
    m1 = a1 > 0.0
    h1 = jnp.maximum(a1, 0.0)

    wo2t = wo2t_ref[...]
    scores = jax.lax.dot_general(h1, wo2t, (((1,), (1,)), ((), ())),
                                 preferred_element_type=jnp.float32)
    scores_ref[...] = scores

    g1 = jnp.where(m1, wo2t, 0.0)
    g0 = jax.lax.dot_general(g1, wo1_ref[...], (((1,), (1,)), ((), ())),
                             preferred_element_type=jnp.float32)
    g0 = jnp.where(m0, g0, 0.0)
    gc = jax.lax.dot_general(g0, wo0_ref[...], (((1,), (1,)), ((), ())),
                             preferred_element_type=jnp.float32)
    gpre = jnp.where(m, gc, 0.0)
    v_ref[...] = -jnp.mean(gpre, axis=0, keepdims=True)


def _stage_b(pre0, pre1, pre2, bcat, wo0, bo0, wo1, bo1, wo2t):
    full = lambda s: pl.BlockSpec(s, lambda: tuple(0 for _ in s))
    return pl.pallas_call(
        _stage_b_body,
        in_specs=[full((MASK, H)), full((MASK, H)), full((MASK, H)),
                  full((1, 3 * H)), full((3 * H, 128)),
                  full((1, 128)), full((128, 32)), full((1, 32)),
                  full((1, 32))],
        out_specs=[full((MASK, 1)), full((1, 3 * H))],
        out_shape=[jax.ShapeDtypeStruct((MASK, 1), jnp.float32),
                   jax.ShapeDtypeStruct((1, 3 * H), jnp.float32)],
    )(pre0, pre1, pre2, bcat, wo0, bo0, wo1, bo1, wo2t)


def _sortable_key(f):
    b = jax.lax.bitcast_convert_type(f, jnp.uint32)
    neg = b >= jnp.uint32(0x80000000)
    return jnp.where(neg, ~b, b | jnp.uint32(0x80000000))


def _stage_c_body(w_ref, v_ref, out_ref, *, fd, k):
    imp = jax.lax.dot_general(v_ref[...], w_ref[...], (((1,), (1,)), ((), ())),
                              preferred_element_type=jnp.float32)
    key = _sortable_key(imp)
    kk = jnp.int32(k)

    def bit_step(i, t):
        cand = t | (jnp.uint32(1) << (jnp.uint32(31) - i.astype(jnp.uint32)))
        cnt = jnp.sum((key >= cand).astype(jnp.int32))
        return jnp.where(cnt >= kk, cand, t)

    kth = jax.lax.fori_loop(0, 32, bit_step, jnp.uint32(0))
    row = (key >= kth).astype(jnp.float32)
    if fd < GRID:
        row = jnp.concatenate(
            [row, jnp.zeros((1, GRID - fd), jnp.float32)], axis=1)
    out_ref[...] = row


def _stage_c(w, v, fd, k):
    full = lambda s: pl.BlockSpec(s, lambda: tuple(0 for _ in s))
    return pl.pallas_call(
        functools.partial(_stage_c_body, fd=fd, k=k),
        in_specs=[full((fd, H)), full((1, H))],
        out_specs=full((1, GRID)),
        out_shape=jax.ShapeDtypeStruct((1, GRID), jnp.float32),
    )(w, v)


def kernel(x, W0, b0, W1, b1, W2, b2, Wo0, bo0, Wo1, bo1, Wo2, bo2):
    x0 = x[0]
    x1 = jax.lax.slice(x, (1, 0, 0), (2, MASK, FEATURE_DIMS[1]))[0]
    x2 = jax.lax.slice(x, (2, 0, 0), (3, MASK, FEATURE_DIMS[2]))[0]
    pre0 = _stage_a(x0, W0)
    pre1 = _stage_a(x1, W1)
    pre2 = _stage_a(x2, W2)

    bcat = jnp.concatenate([b0, b1, b2])[None, :]
    scores, vrow = _stage_b(pre0, pre1, pre2, bcat, Wo0, bo0[None, :],
                            Wo1, bo1[None, :], Wo2.T)
    scores = scores + bo2[None, :]

    ws = (W0, W1, W2)
    rows = [
        _stage_c(ws[i], jax.lax.slice(vrow, (0, i * H), (1, (i + 1) * H)),
                 FEATURE_DIMS[i], UNMASKED[i])
        for i in range(3)
    ]
    mask_opt = jnp.concatenate(rows, axis=0)
    return scores, mask_opt
